# baseline (speedup 1.0000x reference)
import functools

import jax
import jax.numpy as jnp
from jax import lax
from jax.experimental import pallas as pl
from jax.experimental.pallas import tpu as pltpu
from jax.experimental.pallas import tpu_sc as plsc

N = 1_000_000
D = 20
NB = 2_304
NFULL = N // NB
NT = N - NFULL * NB
NW = 32
VECS = NB // 16
VECS_T = NT // 16
BPT = 14


def _compute(xv, ov, cols, nvec, width):
    @plsc.parallel_loop(0, nvec, 1, unroll=8)
    def do_vec(v):
        x16 = xv[pl.ds(v * 16, 16)]
        idx = jnp.minimum(x16, 5)
        idx = idx + jnp.minimum(x16 >> 3, 1)
        idx = idx + jnp.minimum(x16 >> 4, 1)
        idx = idx + jnp.minimum(x16 >> 5, 1)
        idx = idx + jnp.minimum(x16 >> 6, 1)
        for d in range(D):
            ov[d, pl.ds(v * 16, 16)] = jnp.take_along_axis(cols[d], idx,
                                                           axis=0)


def _body(x_hbm, table_hbm, out_hbm, xa, xb, table_v, outa, outb, outt,
          sxa, sxb, soa, sob):
    wid = lax.axis_index("s") * 2 + lax.axis_index("c")
    pltpu.sync_copy(table_hbm, table_v)
    ridx = jnp.minimum(lax.iota(jnp.int32, 16), 9) * D
    cols = [plsc.load_gather(table_v, [ridx + d]) for d in range(D)]

    def base(t):
        return jnp.minimum(wid + t * NW, NFULL - 1) * NB

    def start_x(t, xv, sem):
        pltpu.async_copy(x_hbm.at[pl.ds(base(t), NB)], xv, sem)

    def wait_x(t, xv, sem):
        pltpu.make_async_copy(x_hbm.at[pl.ds(base(t), NB)], xv, sem).wait()

    def start_out(t, ov, sem):
        pltpu.async_copy(ov, out_hbm.at[:, pl.ds(base(t), NB)], sem)

    def wait_out(t, ov, sem):
        pltpu.make_async_copy(ov, out_hbm.at[:, pl.ds(base(t), NB)],
                              sem).wait()

    start_x(0, xa, sxa)
    start_x(1, xb, sxb)

    def do_pair(i, _):
        tA = 2 * i
        tB = 2 * i + 1
        wait_x(tA, xa, sxa)

        @pl.when(i > 0)
        def _():
            wait_out(tA, outa, soa)

        _compute(xa, outa, cols, VECS, NB)
        start_out(tA, outa, soa)

        @pl.when(tA + 2 < BPT)
        def _():
            start_x(tA + 2, xa, sxa)

        wait_x(tB, xb, sxb)

        @pl.when(i > 0)
        def _():
            wait_out(tB, outb, sob)

        _compute(xb, outb, cols, VECS, NB)
        start_out(tB, outb, sob)

        @pl.when(tB + 2 < BPT)
        def _():
            start_x(tB + 2, xb, sxb)

        return 0

    lax.fori_loop(0, BPT // 2, do_pair, 0)
    wait_out(BPT - 2, outa, soa)
    wait_out(BPT - 1, outb, sob)

    @pl.when(wid == NW - 1)
    def _():
        pltpu.sync_copy(x_hbm.at[pl.ds(NFULL * NB, NT)], xa.at[pl.ds(0, NT)])
        _compute(xa, outt, cols, VECS_T, NT)
        pltpu.sync_copy(outt, out_hbm.at[:, pl.ds(NFULL * NB, NT)])


def kernel(x, table):
    mesh = plsc.VectorSubcoreMesh(core_axis_name="c", subcore_axis_name="s")
    f = functools.partial(
        pl.kernel,
        mesh=mesh,
        compiler_params=pltpu.CompilerParams(needs_layout_passes=False),
        out_type=jax.ShapeDtypeStruct((D, N), jnp.float32),
        scratch_types=[
            pltpu.VMEM((NB,), jnp.int32),
            pltpu.VMEM((NB,), jnp.int32),
            pltpu.VMEM((10 * D,), jnp.float32),
            pltpu.VMEM((D, NB), jnp.float32),
            pltpu.VMEM((D, NB), jnp.float32),
            pltpu.VMEM((D, NT), jnp.float32),
            pltpu.SemaphoreType.DMA,
            pltpu.SemaphoreType.DMA,
            pltpu.SemaphoreType.DMA,
            pltpu.SemaphoreType.DMA,
        ],
    )(_body)
    out = f(x, table.reshape(10 * D))
    return out.T


# speedup vs baseline: 1.0801x; 1.0801x over previous
import functools

import jax
import jax.numpy as jnp
from jax import lax
from jax.experimental import pallas as pl
from jax.experimental.pallas import tpu as pltpu
from jax.experimental.pallas import tpu_sc as plsc

N = 1_000_000
D = 20
NB = 2_304
NFULL = N // NB
NT = N - NFULL * NB
NW = 32
VECS = NB // 16
VECS_T = NT // 16
BPT = 14


def _compute(xv, ov, cols, nvec, width):
    @plsc.parallel_loop(0, nvec, 1, unroll=6)
    def do_vec(v):
        x16 = xv[pl.ds(v * 16, 16)]
        idx = jnp.minimum(x16, 5)
        idx = idx + jnp.minimum(x16 >> 3, 1)
        idx = idx + jnp.minimum(x16 >> 4, 1)
        idx = idx + jnp.minimum(x16 >> 5, 1)
        idx = idx + jnp.minimum(x16 >> 6, 1)
        for d in range(D):
            ov[d, pl.ds(v * 16, 16)] = jnp.take_along_axis(cols[d], idx,
                                                           axis=0)


def _body(x_hbm, table_hbm, out_hbm, xa, xb, table_v, outa, outb, outt,
          sxa, sxb, soa, sob):
    wid = lax.axis_index("s") * 2 + lax.axis_index("c")
    pltpu.sync_copy(table_hbm, table_v)
    ridx = jnp.minimum(lax.iota(jnp.int32, 16), 9) * D
    cols = [plsc.load_gather(table_v, [ridx + d]) for d in range(D)]

    def base(t):
        return jnp.minimum(wid + t * NW, NFULL - 1) * NB

    def start_x(t, xv, sem):
        pltpu.async_copy(x_hbm.at[pl.ds(base(t), NB)], xv, sem)

    def wait_x(t, xv, sem):
        pltpu.make_async_copy(x_hbm.at[pl.ds(base(t), NB)], xv, sem).wait()

    def start_out(t, ov, sem):
        pltpu.async_copy(ov, out_hbm.at[:, pl.ds(base(t), NB)], sem)

    def wait_out(t, ov, sem):
        pltpu.make_async_copy(ov, out_hbm.at[:, pl.ds(base(t), NB)],
                              sem).wait()

    start_x(0, xa, sxa)
    start_x(1, xb, sxb)

    def do_pair(i, _):
        tA = 2 * i
        tB = 2 * i + 1
        wait_x(tA, xa, sxa)

        @pl.when(i > 0)
        def _():
            wait_out(tA, outa, soa)

        _compute(xa, outa, cols, VECS, NB)
        start_out(tA, outa, soa)

        @pl.when(tA + 2 < BPT)
        def _():
            start_x(tA + 2, xa, sxa)

        wait_x(tB, xb, sxb)

        @pl.when(i > 0)
        def _():
            wait_out(tB, outb, sob)

        _compute(xb, outb, cols, VECS, NB)
        start_out(tB, outb, sob)

        @pl.when(tB + 2 < BPT)
        def _():
            start_x(tB + 2, xb, sxb)

        return 0

    lax.fori_loop(0, BPT // 2, do_pair, 0)
    wait_out(BPT - 2, outa, soa)
    wait_out(BPT - 1, outb, sob)

    @pl.when(wid == NW - 1)
    def _():
        pltpu.sync_copy(x_hbm.at[pl.ds(NFULL * NB, NT)], xa.at[pl.ds(0, NT)])
        _compute(xa, outt, cols, VECS_T, NT)
        pltpu.sync_copy(outt, out_hbm.at[:, pl.ds(NFULL * NB, NT)])


def kernel(x, table):
    mesh = plsc.VectorSubcoreMesh(core_axis_name="c", subcore_axis_name="s")
    f = functools.partial(
        pl.kernel,
        mesh=mesh,
        compiler_params=pltpu.CompilerParams(needs_layout_passes=False),
        out_type=jax.ShapeDtypeStruct((D, N), jnp.float32),
        scratch_types=[
            pltpu.VMEM((NB,), jnp.int32),
            pltpu.VMEM((NB,), jnp.int32),
            pltpu.VMEM((10 * D,), jnp.float32),
            pltpu.VMEM((D, NB), jnp.float32),
            pltpu.VMEM((D, NB), jnp.float32),
            pltpu.VMEM((D, NT), jnp.float32),
            pltpu.SemaphoreType.DMA,
            pltpu.SemaphoreType.DMA,
            pltpu.SemaphoreType.DMA,
            pltpu.SemaphoreType.DMA,
        ],
    )(_body)
    out = f(x, table.reshape(10 * D))
    return out.T
